# baseline (speedup 1.0000x reference)
import jax
import jax.numpy as jnp
from jax.experimental import pallas as pl

EPS = 1e-5


def _conv1x1(x, W):
    return jnp.einsum('oc,bcl->bol', W, x)


def _bn_relu(x, g, b):
    mean = jnp.mean(x, axis=(0, 2), keepdims=True)
    var = jnp.var(x, axis=(0, 2), keepdims=True)
    y = (x - mean) / jnp.sqrt(var + EPS) * g[None, :, None] + b[None, :, None]
    return jax.nn.relu(y)


def _index_points(points, idx):
    B = points.shape[0]
    bidx = jnp.arange(B).reshape((B,) + (1,) * (idx.ndim - 1))
    return points[bidx, idx]


def _fps(xyz, npoint):
    B, N, _ = xyz.shape
    def step(carry, _):
        distance, farthest = carry
        centroid = xyz[jnp.arange(B), farthest][:, None, :]
        d = jnp.sum((xyz - centroid) ** 2, axis=-1)
        distance = jnp.minimum(distance, d)
        new_farthest = jnp.argmax(distance, axis=-1).astype(jnp.int32)
        return (distance, new_farthest), farthest
    init = (jnp.full((B, N), 1e10, dtype=xyz.dtype), jnp.zeros((B,), dtype=jnp.int32))
    _, idxs = jax.lax.scan(step, init, None, length=npoint)
    return jnp.transpose(idxs)


def _sqdist(src, dst):
    d = -2.0 * jnp.einsum('bnc,bmc->bnm', src, dst)
    d = d + jnp.sum(src ** 2, -1)[:, :, None]
    d = d + jnp.sum(dst ** 2, -1)[:, None, :]
    return d


def _knn(k, xyz, new_xyz):
    sqrdists = _sqdist(new_xyz, xyz)
    _, idx = jax.lax.top_k(-sqrdists, k)
    return idx


def _sg_apply(x, fps_idx, knn_idx, W1, g1, b1, W2, g2, b2):
    feats = jnp.transpose(x, (0, 2, 1))
    new_features = _index_points(feats, fps_idx)
    grouped = _index_points(feats, knn_idx)
    normed = grouped - new_features[:, :, None, :]
    agg = jnp.concatenate([normed, jnp.broadcast_to(new_features[:, :, None, :], normed.shape)], axis=-1)
    b_, s_, k_, d_ = agg.shape
    nf = jnp.transpose(agg, (0, 1, 3, 2)).reshape(b_ * s_, d_, k_)
    nf = _bn_relu(_conv1x1(nf, W1), g1, b1)
    nf = _bn_relu(_conv1x1(nf, W2), g2, b2)
    nf = jnp.max(nf, axis=2)
    return jnp.transpose(nf.reshape(b_, s_, -1), (0, 2, 1))


def _identity_kernel(x_ref, o_ref):
    o_ref[...] = x_ref[...]


def kernel(x, W1, g1, b1, W2, g2, b2, sg1_W1, sg1_g1, sg1_b1, sg1_W2, sg1_g2, sg1_b2, sg2_W1, sg2_g1, sg2_b1, sg2_W2, sg2_g2, sg2_b2):
    xyz = jnp.transpose(x, (0, 2, 1))
    fps_idx1 = _fps(xyz, 512)
    xyz1 = _index_points(xyz, fps_idx1)
    knn_idx1 = _knn(32, xyz, xyz1)
    fps_idx2 = _fps(xyz1, 256)
    xyz2 = _index_points(xyz1, fps_idx2)
    knn_idx2 = _knn(32, xyz1, xyz2)
    f = _bn_relu(_conv1x1(x, W1), g1, b1)
    f = _bn_relu(_conv1x1(f, W2), g2, b2)
    f1 = _sg_apply(f, fps_idx1, knn_idx1, sg1_W1, sg1_g1, sg1_b1, sg1_W2, sg1_g2, sg1_b2)
    f2 = _sg_apply(f1, fps_idx2, knn_idx2, sg2_W1, sg2_g1, sg2_b1, sg2_W2, sg2_g2, sg2_b2)
    out = pl.pallas_call(
        _identity_kernel,
        out_shape=jax.ShapeDtypeStruct(f2.shape, f2.dtype),
    )(f2)
    return out


# speedup vs baseline: 8.5233x; 8.5233x over previous
import functools

import jax
import jax.numpy as jnp
from jax import lax
from jax.experimental import pallas as pl
from jax.experimental.pallas import tpu as pltpu
from jax.experimental.pallas import tpu_sc as plsc

EPS = 1e-5
B = 16
N1 = 1024
S1 = 512
S2 = 256
K = 32
F32 = jnp.float32
I32 = jnp.int32


def _fps_body(npoint, px_ref, py_ref, pz_ref, idx_ref, cx_ref, cy_ref, cz_ref):
    X = px_ref[...]
    Y = py_ref[...]
    Z = pz_ref[...]
    b, n = X.shape
    lane = lax.broadcasted_iota(I32, (b, n), 1)

    def step(i, carry):
        dist, far = carry
        mask = lane == far
        cx = jnp.sum(jnp.where(mask, X, 0.0), axis=1, keepdims=True)
        cy = jnp.sum(jnp.where(mask, Y, 0.0), axis=1, keepdims=True)
        cz = jnp.sum(jnp.where(mask, Z, 0.0), axis=1, keepdims=True)
        idx_ref[pl.ds(i, 1), :] = far.reshape(1, b)
        cx_ref[pl.ds(i, 1), :] = cx.reshape(1, b)
        cy_ref[pl.ds(i, 1), :] = cy.reshape(1, b)
        cz_ref[pl.ds(i, 1), :] = cz.reshape(1, b)
        d = (X - cx) ** 2 + (Y - cy) ** 2 + (Z - cz) ** 2
        dist = jnp.minimum(dist, d)
        m = jnp.max(dist, axis=1, keepdims=True)
        t = jnp.where(dist == m, lane, n)
        far = jnp.min(t, axis=1, keepdims=True).astype(I32)
        return dist, far

    init = (jnp.full((b, n), 1e10, dtype=F32), jnp.zeros((b, 1), dtype=I32))
    lax.fori_loop(0, npoint, step, init)


def _fps(px, py, pz, npoint):
    b, n = px.shape
    out_shapes = (
        jax.ShapeDtypeStruct((npoint, b), I32),
        jax.ShapeDtypeStruct((npoint, b), F32),
        jax.ShapeDtypeStruct((npoint, b), F32),
        jax.ShapeDtypeStruct((npoint, b), F32),
    )
    return pl.pallas_call(
        functools.partial(_fps_body, npoint),
        out_shape=out_shapes,
    )(px, py, pz)


def _knn_body(n_per_batch, q_ref, pt_ref, idx_ref):
    q = q_ref[...]
    p3 = pt_ref[0]
    s, _ = q.shape
    n = p3.shape[1]
    qn = jnp.sum(q * q, axis=1, keepdims=True)
    pn = jnp.sum(p3 * p3, axis=0, keepdims=True)
    D = -2.0 * jnp.dot(q, p3, preferred_element_type=F32)
    D = D + qn
    D = D + pn
    lane = lax.broadcasted_iota(I32, (s, n), 1)
    colid = lax.broadcasted_iota(I32, (s, K), 1)
    INF = jnp.float32(3.0e38)

    def it(i, carry):
        D, acc = carry
        m = jnp.min(D, axis=1, keepdims=True)
        t = jnp.where(D == m, lane, n)
        idx = jnp.min(t, axis=1, keepdims=True).astype(I32)
        D = jnp.where(t == idx, INF, D)
        acc = jnp.where(colid == i, idx, acc)
        return D, acc

    _, acc = lax.fori_loop(0, K, it, (D, jnp.zeros((s, K), I32)))
    bidx = pl.program_id(0)
    idx_ref[...] = acc + bidx * n_per_batch


def _knn(qrows, pt, s_per_batch, n_per_batch):
    return pl.pallas_call(
        functools.partial(_knn_body, n_per_batch),
        grid=(B,),
        in_specs=[
            pl.BlockSpec((s_per_batch, 3), lambda i: (i, 0)),
            pl.BlockSpec((1, 3, n_per_batch), lambda i: (i, 0, 0)),
        ],
        out_specs=pl.BlockSpec((s_per_batch, K), lambda i: (i, 0)),
        out_shape=jax.ShapeDtypeStruct((B * s_per_batch, K), I32),
        compiler_params=pltpu.CompilerParams(
            dimension_semantics=("arbitrary",)),
    )(qrows, pt)


def _d0_body(x_ref, w1_ref, g1_ref, b1_ref, w2_ref, g2_ref, b2_ref,
             wa_ref, wfull_ref, g_out_ref, gb_out_ref):
    x = x_ref[...]
    h = lax.dot_general(x, w1_ref[...], (((1,), (1,)), ((), ())),
                        preferred_element_type=F32)
    mean = jnp.mean(h, axis=0, keepdims=True)
    var = jnp.mean((h - mean) ** 2, axis=0, keepdims=True)
    h = (h - mean) / jnp.sqrt(var + EPS) * g1_ref[...] + b1_ref[...]
    h = jnp.maximum(h, 0.0)
    h = lax.dot_general(h, w2_ref[...], (((1,), (1,)), ((), ())),
                        preferred_element_type=F32)
    mean = jnp.mean(h, axis=0, keepdims=True)
    var = jnp.mean((h - mean) ** 2, axis=0, keepdims=True)
    h = (h - mean) / jnp.sqrt(var + EPS) * g2_ref[...] + b2_ref[...]
    h = jnp.maximum(h, 0.0)
    wa = wa_ref[...]
    wb = wfull_ref[...][:, 64:]
    g_out_ref[...] = lax.dot_general(h, wa, (((1,), (1,)), ((), ())),
                                     preferred_element_type=F32)
    gb_out_ref[...] = lax.dot_general(h, wb - wa, (((1,), (1,)), ((), ())),
                                      preferred_element_type=F32)


def _d0(xrows, W1, g1, b1, W2, g2, b2, sg1_W1):
    bn = xrows.shape[0]
    return pl.pallas_call(
        _d0_body,
        out_shape=(
            jax.ShapeDtypeStruct((bn, 128), F32),
            jax.ShapeDtypeStruct((bn, 128), F32),
        ),
    )(xrows, W1, g1.reshape(1, -1), b1.reshape(1, -1), W2,
      g2.reshape(1, -1), b2.reshape(1, -1), sg1_W1[:, :64], sg1_W1)


def _sc_gather(knn_idx2d, fps_idx2d, gtab, gbtab, width):
    cg = knn_idx2d.shape[0]
    cc = fps_idx2d.shape[0]
    n_workers = 32
    per_g = cg // n_workers
    per_c = cc // n_workers
    mesh = plsc.VectorSubcoreMesh(core_axis_name="c", subcore_axis_name="s")

    @functools.partial(
        pl.kernel,
        mesh=mesh,
        out_type=(
            jax.ShapeDtypeStruct((cg * 128, width), F32),
            jax.ShapeDtypeStruct((cc * 128, width), F32),
        ),
        scratch_types=[
            pltpu.VMEM((128,), I32),
            pltpu.VMEM((128, width), F32),
            pltpu.SemaphoreType.DMA,
        ],
    )
    def k(knn_hbm, fps_hbm, g_hbm, gb_hbm, outg_hbm, outc_hbm,
          idx_v, rows_v, sem):
        cid = lax.axis_index("c")
        sid = lax.axis_index("s")
        wid = sid * 2 + cid

        def body(i, _):
            ci = wid * per_g + i
            pltpu.sync_copy(knn_hbm.at[ci], idx_v)
            pltpu.async_copy(g_hbm.at[idx_v], rows_v, sem).wait()
            pltpu.sync_copy(rows_v, outg_hbm.at[pl.ds(ci * 128, 128)])
            return 0

        lax.fori_loop(0, per_g, body, 0)

        def body_c(i, _):
            ci = wid * per_c + i
            pltpu.sync_copy(fps_hbm.at[ci], idx_v)
            pltpu.async_copy(gb_hbm.at[idx_v], rows_v, sem).wait()
            pltpu.sync_copy(rows_v, outc_hbm.at[pl.ds(ci * 128, 128)])
            return 0

        lax.fori_loop(0, per_c, body_c, 0)

    return k(knn_idx2d, fps_idx2d, gtab, gbtab)


_GB = 64


def _p1_body(nsteps, minv, g_ref, c_ref, mean_ref, var_ref, acc):
    y = g_ref[...].reshape(_GB, K, g_ref.shape[1])
    y = y + c_ref[...][:, None, :]
    s = jnp.sum(y, axis=(0, 1), keepdims=True)[0]
    sq = jnp.sum(y * y, axis=(0, 1), keepdims=True)[0]

    @pl.when(pl.program_id(0) == 0)
    def _():
        acc[...] = jnp.zeros_like(acc)

    acc[0:1, :] += s
    acc[1:2, :] += sq

    @pl.when(pl.program_id(0) == nsteps - 1)
    def _():
        mean = acc[0:1, :] * minv
        mean_ref[...] = mean
        var_ref[...] = acc[1:2, :] * minv - mean * mean


def _p1(grouped, crows, width, n_groups):
    nsteps = n_groups // _GB
    minv = 1.0 / (n_groups * K)
    return pl.pallas_call(
        functools.partial(_p1_body, nsteps, minv),
        grid=(nsteps,),
        in_specs=[
            pl.BlockSpec((_GB * K, width), lambda i: (i, 0)),
            pl.BlockSpec((_GB, width), lambda i: (i, 0)),
        ],
        out_specs=[
            pl.BlockSpec((1, width), lambda i: (0, 0)),
            pl.BlockSpec((1, width), lambda i: (0, 0)),
        ],
        out_shape=(
            jax.ShapeDtypeStruct((1, width), F32),
            jax.ShapeDtypeStruct((1, width), F32),
        ),
        scratch_shapes=[pltpu.VMEM((2, width), F32)],
        compiler_params=pltpu.CompilerParams(
            dimension_semantics=("arbitrary",)),
    )(grouped, crows)


def _p2_body(nsteps, g_ref, c_ref, mean_ref, var_ref, gam_ref, bet_ref,
             w2_ref, gmax_ref, stats_ref, acc):
    w = g_ref.shape[1]
    y = g_ref[...].reshape(_GB, K, w) + c_ref[...][:, None, :]
    scale = gam_ref[...] / jnp.sqrt(var_ref[...] + EPS)
    z = jnp.maximum((y - mean_ref[...][:, None, :]) * scale[:, None, :]
                    + bet_ref[...][:, None, :], 0.0)
    z2 = z.reshape(_GB * K, w)
    y2 = lax.dot_general(z2, w2_ref[...], (((1,), (1,)), ((), ())),
                         preferred_element_type=F32)
    o = y2.shape[1]
    s = jnp.sum(y2, axis=0, keepdims=True)
    sq = jnp.sum(y2 * y2, axis=0, keepdims=True)

    @pl.when(pl.program_id(0) == 0)
    def _():
        acc[...] = jnp.zeros_like(acc)

    acc[0:1, :] += s
    acc[1:2, :] += sq
    gmax_ref[...] = jnp.max(y2.reshape(_GB, K, o), axis=1)

    @pl.when(pl.program_id(0) == nsteps - 1)
    def _():
        stats_ref[...] = acc[...]


def _p2(grouped, crows, mean, var, gam, bet, W2, width, n_groups):
    nsteps = n_groups // _GB
    return pl.pallas_call(
        functools.partial(_p2_body, nsteps),
        grid=(nsteps,),
        in_specs=[
            pl.BlockSpec((_GB * K, width), lambda i: (i, 0)),
            pl.BlockSpec((_GB, width), lambda i: (i, 0)),
            pl.BlockSpec((1, width), lambda i: (0, 0)),
            pl.BlockSpec((1, width), lambda i: (0, 0)),
            pl.BlockSpec((1, width), lambda i: (0, 0)),
            pl.BlockSpec((1, width), lambda i: (0, 0)),
            pl.BlockSpec((width, width), lambda i: (0, 0)),
        ],
        out_specs=[
            pl.BlockSpec((_GB, width), lambda i: (i, 0)),
            pl.BlockSpec((2, width), lambda i: (0, 0)),
        ],
        out_shape=(
            jax.ShapeDtypeStruct((n_groups, width), F32),
            jax.ShapeDtypeStruct((2, width), F32),
        ),
        scratch_shapes=[pltpu.VMEM((2, width), F32)],
        compiler_params=pltpu.CompilerParams(
            dimension_semantics=("arbitrary",)),
    )(grouped, crows, mean, var, gam, bet, W2)


def _p3_body(minv, has_tables, gmax_ref, stats_ref, gam_ref, bet_ref,
             wa_ref, wfull_ref, out_ref, *table_refs):
    mean = stats_ref[0:1, :] * minv
    var = stats_ref[1:2, :] * minv - mean * mean
    scale = gam_ref[...] / jnp.sqrt(var + EPS)
    out = jnp.maximum((gmax_ref[...] - mean) * scale + bet_ref[...], 0.0)
    out_ref[...] = out
    if has_tables:
        half = wa_ref.shape[1]
        wa = wa_ref[...]
        wb = wfull_ref[...][:, half:]
        table_refs[0][...] = lax.dot_general(
            out, wa, (((1,), (1,)), ((), ())), preferred_element_type=F32)
        table_refs[1][...] = lax.dot_general(
            out, wb - wa, (((1,), (1,)), ((), ())), preferred_element_type=F32)


def _p3(gmax, stats, gam, bet, width, n_groups, next_w1=None):
    minv = 1.0 / (n_groups * K)
    has_tables = next_w1 is not None
    nb = 2048
    nsteps = n_groups // nb if n_groups >= nb else 1
    nb = min(nb, n_groups)
    out_shapes = [jax.ShapeDtypeStruct((n_groups, width), F32)]
    out_specs = [pl.BlockSpec((nb, width), lambda i: (i, 0))]
    if has_tables:
        ow = next_w1.shape[0]
        out_shapes += [jax.ShapeDtypeStruct((n_groups, ow), F32)] * 2
        out_specs += [pl.BlockSpec((nb, ow), lambda i: (i, 0))] * 2
        wa = next_w1[:, :width]
        wfull = next_w1
    else:
        wa = jnp.zeros((8, width), F32)
        wfull = jnp.zeros((8, 2 * width), F32)
    res = pl.pallas_call(
        functools.partial(_p3_body, minv, has_tables),
        grid=(nsteps,),
        in_specs=[
            pl.BlockSpec((nb, width), lambda i: (i, 0)),
            pl.BlockSpec((2, width), lambda i: (0, 0)),
            pl.BlockSpec((1, width), lambda i: (0, 0)),
            pl.BlockSpec((1, width), lambda i: (0, 0)),
            pl.BlockSpec(wa.shape, lambda i: (0, 0)),
            pl.BlockSpec(wfull.shape, lambda i: (0, 0)),
        ],
        out_specs=out_specs,
        out_shape=tuple(out_shapes),
        compiler_params=pltpu.CompilerParams(
            dimension_semantics=("arbitrary",)),
    )(gmax, stats, gam, bet, wa, wfull)
    return res


def kernel(x, W1, g1, b1, W2, g2, b2, sg1_W1, sg1_g1, sg1_b1, sg1_W2,
           sg1_g2, sg1_b2, sg2_W1, sg2_g1, sg2_b1, sg2_W2, sg2_g2, sg2_b2):
    px = x[:, 0, :]
    py = x[:, 1, :]
    pz = x[:, 2, :]

    idx1, cx1, cy1, cz1 = _fps(px, py, pz, S1)
    q1 = jnp.stack([cx1.T, cy1.T, cz1.T], axis=-1).reshape(B * S1, 3)
    fps1_flat = (idx1.T + jnp.arange(B, dtype=I32)[:, None] * N1).reshape(-1)

    knn1 = _knn(q1, x, S1, N1)

    px1, py1, pz1 = cx1.T, cy1.T, cz1.T
    idx2, cx2, cy2, cz2 = _fps(px1, py1, pz1, S2)
    q2 = jnp.stack([cx2.T, cy2.T, cz2.T], axis=-1).reshape(B * S2, 3)
    fps2_flat = (idx2.T + jnp.arange(B, dtype=I32)[:, None] * S1).reshape(-1)
    pt2 = jnp.stack([px1, py1, pz1], axis=1)
    knn2 = _knn(q2, pt2, S2, S1)

    xrows = jnp.transpose(x, (0, 2, 1)).reshape(B * N1, 3)
    G1, G1b = _d0(xrows, W1, g1, b1, W2, g2, b2, sg1_W1)

    ng1 = B * S1
    grouped1, c1 = _sc_gather(knn1.reshape(-1, 128), fps1_flat.reshape(-1, 128),
                              G1, G1b, 128)
    mean1, var1 = _p1(grouped1, c1, 128, ng1)
    gmax1, stats1 = _p2(grouped1, c1, mean1, var1, sg1_g1.reshape(1, -1),
                        sg1_b1.reshape(1, -1), sg1_W2, 128, ng1)
    _f1, G2, G2b = _p3(gmax1, stats1, sg1_g2.reshape(1, -1),
                       sg1_b2.reshape(1, -1), 128, ng1, next_w1=sg2_W1)

    ng2 = B * S2
    grouped2, c2 = _sc_gather(knn2.reshape(-1, 128), fps2_flat.reshape(-1, 128),
                              G2, G2b, 256)
    mean2, var2 = _p1(grouped2, c2, 256, ng2)
    gmax2, stats2 = _p2(grouped2, c2, mean2, var2, sg2_g1.reshape(1, -1),
                        sg2_b1.reshape(1, -1), sg2_W2, 256, ng2)
    (f2rows,) = _p3(gmax2, stats2, sg2_g2.reshape(1, -1),
                    sg2_b2.reshape(1, -1), 256, ng2)

    return jnp.transpose(f2rows.reshape(B, S2, 256), (0, 2, 1))
